# baseline (speedup 1.0000x reference)
import functools

import jax
import jax.numpy as jnp
import numpy as np
from jax import lax
from jax.experimental import pallas as pl
from jax.experimental.pallas import tpu as pltpu
from jax.experimental.pallas import tpu_sc as plsc

B, L = 16384, 50
JOBS, MACHINES, MAXOPS, D = 100000, 1000, 200, 64
N = B * L

NC, NS = 2, 16
NW = NC * NS
TPW = N // NW
T = 128
CHUNKS = TPW // T


def _project_job_table(job_table, W_proj):
    blk = 4000

    def body(jt, w, o):
        o[...] = jnp.dot(jt[...], w[0:D, :],
                         preferred_element_type=jnp.float32).astype(jnp.bfloat16)

    return pl.pallas_call(
        body,
        grid=(JOBS // blk,),
        in_specs=[
            pl.BlockSpec((blk, D), lambda i: (i, 0)),
            pl.BlockSpec((4 * D, D), lambda i: (0, 0)),
        ],
        out_specs=pl.BlockSpec((blk, D), lambda i: (i, 0)),
        out_shape=jax.ShapeDtypeStruct((JOBS, D), jnp.bfloat16),
    )(job_table, W_proj)


def _project_small_tables(machine_table, seq_table, W_perm, W_proj, W_time,
                          b_time, b_proj_perm):
    def body(mt, st, wp, w, wt, bt, bpp, pm_o, ps_o, v_o):
        c = jnp.dot(bt[...], wp[3 * D:4 * D, :],
                    preferred_element_type=jnp.float32) + bpp[...]
        pm_o[...] = (jnp.dot(mt[...], wp[D:2 * D, :],
                             preferred_element_type=jnp.float32)
                     + c).astype(jnp.bfloat16)
        ps_o[...] = jnp.dot(st[...], wp[2 * D:3 * D, :],
                            preferred_element_type=jnp.float32
                            ).astype(jnp.bfloat16)
        v_o[...] = jnp.dot(wt[...], w[3 * D:4 * D, :],
                           preferred_element_type=jnp.float32)

    return pl.pallas_call(
        body,
        out_shape=(
            jax.ShapeDtypeStruct((MACHINES, D), jnp.bfloat16),
            jax.ShapeDtypeStruct((MAXOPS, D), jnp.bfloat16),
            jax.ShapeDtypeStruct((1, D), jnp.float32),
        ),
    )(machine_table, seq_table, W_perm, W_proj, W_time,
      b_time.reshape(1, D), b_proj_perm.reshape(1, D))


def _sc_gather_combine(sidx, timef, pjob, pmach, pseq, vrow, npart):
    mesh = plsc.VectorSubcoreMesh(core_axis_name="c", subcore_axis_name="s")
    NP = N // npart
    GPW = (NP // NW) // 128
    NCHUNK = GPW // 2

    @functools.partial(
        pl.kernel,
        out_type=jax.ShapeDtypeStruct((NP, D), jnp.float32),
        mesh=mesh,
        scratch_types=[
            pltpu.VMEM((3, 2, 128), jnp.int32),
            pltpu.VMEM((3, 2, 128), jnp.int32),
            pltpu.VMEM((2, 128), jnp.float32),
            pltpu.VMEM((2, 128), jnp.float32),
            pltpu.VMEM((256, D), jnp.bfloat16),
            pltpu.VMEM((256, D), jnp.bfloat16),
            pltpu.VMEM((256, D), jnp.bfloat16),
            pltpu.VMEM((256, D), jnp.bfloat16),
            pltpu.VMEM((256, D), jnp.bfloat16),
            pltpu.VMEM((256, D), jnp.bfloat16),
            pltpu.VMEM((256, D), jnp.float32),
            pltpu.VMEM((256, D), jnp.float32),
            pltpu.VMEM((D,), jnp.float32),
            pltpu.SemaphoreType.DMA,
            pltpu.SemaphoreType.DMA,
            pltpu.SemaphoreType.DMA,
            pltpu.SemaphoreType.DMA,
            pltpu.SemaphoreType.DMA,
            pltpu.SemaphoreType.DMA,
            pltpu.SemaphoreType.DMA,
            pltpu.SemaphoreType.DMA,
        ],
        compiler_params=pltpu.CompilerParams(use_tc_tiling_on_sc=False,
                                             needs_layout_passes=False),
    )
    def k(sidx_h, timef_h, pjob_h, pmach_h, pseq_h, vrow_h, out_h,
          idx0, idx1, tb0, tb1, bufj0, bufj1, bufm0, bufm1, bufs0, bufs1,
          ob0, ob1, vbuf, sem0, sem1, semo0, semo1, semi0, semi1, semt0, semt1):
        wid = lax.axis_index("s") * NC + lax.axis_index("c")
        pltpu.sync_copy(vrow_h, vbuf)
        vregs = [vbuf[pl.ds(r * 16, 16)] for r in range(D // 16)]
        idx = (idx0, idx1)
        tbuf = (tb0, tb1)
        bufj = (bufj0, bufj1)
        bufm = (bufm0, bufm1)
        bufs = (bufs0, bufs1)
        outb = (ob0, ob1)
        sems = (sem0, sem1)
        semo = (semo0, semo1)
        semi = (semi0, semi1)
        semt = (semt0, semt1)
        grp0 = wid * GPW

        def prefetch_idx(s, g):
            pltpu.async_copy(sidx_h.at[:, pl.ds(grp0 + g * 2, 2), :],
                             idx[s], semi[s])

        def prefetch_t(s, g):
            pltpu.async_copy(timef_h.at[pl.ds(grp0 + g * 2, 2)],
                             tbuf[s], semt[s])

        def fire(s, g):
            pltpu.make_async_copy(sidx_h.at[:, pl.ds(grp0 + g * 2, 2), :],
                                  idx[s], semi[s]).wait()
            for j in range(2):
                dst = pl.ds(j * 128, 128)
                pltpu.async_copy(pjob_h.at[idx[s].at[0, j]],
                                 bufj[s].at[dst], sems[s])
                pltpu.async_copy(pmach_h.at[idx[s].at[1, j]],
                                 bufm[s].at[dst], sems[s])
                pltpu.async_copy(pseq_h.at[idx[s].at[2, j]],
                                 bufs[s].at[dst], sems[s])

        def drain(s):
            for j in range(2):
                dst = pl.ds(j * 128, 128)
                pltpu.make_async_copy(pjob_h.at[idx[s].at[0, j]],
                                      bufj[s].at[dst], sems[s]).wait()
                pltpu.make_async_copy(pmach_h.at[idx[s].at[1, j]],
                                      bufm[s].at[dst], sems[s]).wait()
                pltpu.make_async_copy(pseq_h.at[idx[s].at[2, j]],
                                      bufs[s].at[dst], sems[s]).wait()

        def combine_store(s, g):
            pltpu.make_async_copy(timef_h.at[pl.ds(grp0 + g * 2, 2)],
                                  tbuf[s], semt[s]).wait()

            @pl.when(g >= 2)
            def _():
                pltpu.make_async_copy(
                    outb[s], out_h.at[pl.ds(0, 256)], semo[s]).wait()

            @pl.loop(0, 16)
            def grp(gg):
                tw = tbuf[s][gg // 8, pl.ds((gg % 8) * 16, 16)]
                for t in range(16):
                    tok = gg * 16 + t
                    st = lax.gather(
                        tw, jnp.full((16, 1), t, jnp.int32),
                        lax.GatherDimensionNumbers(
                            offset_dims=(), collapsed_slice_dims=(0,),
                            start_index_map=(0,)),
                        slice_sizes=(1,),
                        mode=lax.GatherScatterMode.PROMISE_IN_BOUNDS)
                    for q in range(2):
                        xj = bufj[s][tok, pl.ds(q * 32, 32)]
                        xm = bufm[s][tok, pl.ds(q * 32, 32)]
                        xs = bufs[s][tok, pl.ds(q * 32, 32)]
                        aj, bj = plsc.unpack(
                            xj, format=plsc.PackFormat.INTERLEAVED)
                        am, bm_ = plsc.unpack(
                            xm, format=plsc.PackFormat.INTERLEAVED)
                        as_, bs_ = plsc.unpack(
                            xs, format=plsc.PackFormat.INTERLEAVED)
                        outb[s][tok, pl.ds(q * 32, 16)] = (
                            aj + am + as_ + st * vregs[2 * q])
                        outb[s][tok, pl.ds(q * 32 + 16, 16)] = (
                            bj + bm_ + bs_ + st * vregs[2 * q + 1])

            pltpu.async_copy(
                outb[s], out_h.at[pl.ds(wid * GPW * 128 + g * 256, 256)],
                semo[s])

        prefetch_idx(0, 0)
        prefetch_t(0, 0)
        fire(0, 0)
        prefetch_idx(1, 1)
        prefetch_t(1, 1)

        @pl.loop(0, NCHUNK, step=2)
        def outer(g):
            drain(0)
            fire(1, g + 1)

            @pl.when(g + 2 < NCHUNK)
            def _():
                prefetch_idx(0, g + 2)
            combine_store(0, g)

            @pl.when(g + 2 < NCHUNK)
            def _():
                prefetch_t(0, g + 2)
            drain(1)

            @pl.when(g + 2 < NCHUNK)
            def _():
                fire(0, g + 2)

            @pl.when(g + 3 < NCHUNK)
            def _():
                prefetch_idx(1, g + 3)
            combine_store(1, g + 1)

            @pl.when(g + 3 < NCHUNK)
            def _():
                prefetch_t(1, g + 3)

        pltpu.make_async_copy(ob0, out_h.at[pl.ds(0, 256)], semo0).wait()
        pltpu.make_async_copy(ob1, out_h.at[pl.ds(0, 256)], semo1).wait()

    return k(sidx, timef, pjob, pmach, pseq, vrow)


def kernel(job, machine, sequence, time, job_table, machine_table, seq_table,
           W_time, b_time, W_proj, b_proj):
    perm = np.empty(D, np.int32)
    for q_ in range(2):
        for i_ in range(16):
            perm[q_ * 32 + 2 * i_] = q_ * 32 + i_
            perm[q_ * 32 + 2 * i_ + 1] = q_ * 32 + 16 + i_
    W_perm = W_proj[:, perm]
    pjob = _project_job_table(job_table, W_perm)
    pmach, pseq, vrow = _project_small_tables(
        machine_table, seq_table, W_perm, W_proj, W_time, b_time,
        b_proj[perm])
    sidx = jnp.stack([
        job.reshape(N).astype(jnp.int32),
        machine.reshape(N).astype(jnp.int32),
        sequence.reshape(N).astype(jnp.int32),
    ]).reshape(3, N // 128, 128)
    timef = time.reshape(N // 128, 128).astype(jnp.float32)
    npart = 2
    gp = (N // 128) // npart
    parts = [
        _sc_gather_combine(
            sidx[:, p * gp:(p + 1) * gp], timef[p * gp:(p + 1) * gp],
            pjob, pmach, pseq, vrow.reshape(D), npart
        ).reshape(B // npart, L, D)
        for p in range(npart)
    ]
    return jnp.concatenate(parts, axis=0)


# speedup vs baseline: 1.0181x; 1.0181x over previous
import functools

import jax
import jax.numpy as jnp
import numpy as np
from jax import lax
from jax.experimental import pallas as pl
from jax.experimental.pallas import tpu as pltpu
from jax.experimental.pallas import tpu_sc as plsc

B, L = 16384, 50
JOBS, MACHINES, MAXOPS, D = 100000, 1000, 200, 64
N = B * L

NC, NS = 2, 16
NW = NC * NS
TPW = N // NW
T = 128
CHUNKS = TPW // T


def _project_job_table(job_table, W_proj):
    blk = 4000

    def body(jt, w, o):
        o[...] = jnp.dot(jt[...], w[0:D, :],
                         preferred_element_type=jnp.float32).astype(jnp.bfloat16)

    return pl.pallas_call(
        body,
        grid=(JOBS // blk,),
        in_specs=[
            pl.BlockSpec((blk, D), lambda i: (i, 0)),
            pl.BlockSpec((4 * D, D), lambda i: (0, 0)),
        ],
        out_specs=pl.BlockSpec((blk, D), lambda i: (i, 0)),
        out_shape=jax.ShapeDtypeStruct((JOBS, D), jnp.bfloat16),
    )(job_table, W_proj)


def _project_small_tables(machine_table, seq_table, W_perm, W_proj, W_time,
                          b_time, b_proj_perm):
    def body(mt, st, wp, w, wt, bt, bpp, pm_o, ps_o, v_o):
        c = jnp.dot(bt[...], wp[3 * D:4 * D, :],
                    preferred_element_type=jnp.float32) + bpp[...]
        pm_o[...] = (jnp.dot(mt[...], wp[D:2 * D, :],
                             preferred_element_type=jnp.float32)
                     + c).astype(jnp.bfloat16)
        ps_o[...] = jnp.dot(st[...], wp[2 * D:3 * D, :],
                            preferred_element_type=jnp.float32
                            ).astype(jnp.bfloat16)
        v_o[...] = jnp.dot(wt[...], w[3 * D:4 * D, :],
                           preferred_element_type=jnp.float32)

    return pl.pallas_call(
        body,
        out_shape=(
            jax.ShapeDtypeStruct((MACHINES, D), jnp.bfloat16),
            jax.ShapeDtypeStruct((MAXOPS, D), jnp.bfloat16),
            jax.ShapeDtypeStruct((1, D), jnp.float32),
        ),
    )(machine_table, seq_table, W_perm, W_proj, W_time,
      b_time.reshape(1, D), b_proj_perm.reshape(1, D))


def _sc_gather_combine(sidx, timef, pjob, pmach, pseq, vrow):
    mesh = plsc.VectorSubcoreMesh(core_axis_name="c", subcore_axis_name="s")
    GPW = TPW // 128
    NCHUNK = GPW // 2

    @functools.partial(
        pl.kernel,
        out_type=jax.ShapeDtypeStruct((N, D), jnp.float32),
        mesh=mesh,
        scratch_types=[
            pltpu.VMEM((3, 2, 128), jnp.int32),
            pltpu.VMEM((3, 2, 128), jnp.int32),
            pltpu.VMEM((2, 128), jnp.float32),
            pltpu.VMEM((2, 128), jnp.float32),
            pltpu.VMEM((256, D), jnp.bfloat16),
            pltpu.VMEM((256, D), jnp.bfloat16),
            pltpu.VMEM((256, D), jnp.bfloat16),
            pltpu.VMEM((256, D), jnp.bfloat16),
            pltpu.VMEM((256, D), jnp.bfloat16),
            pltpu.VMEM((256, D), jnp.bfloat16),
            pltpu.VMEM((256, D), jnp.float32),
            pltpu.VMEM((256, D), jnp.float32),
            pltpu.VMEM((D,), jnp.float32),
            pltpu.SemaphoreType.DMA,
            pltpu.SemaphoreType.DMA,
            pltpu.SemaphoreType.DMA,
            pltpu.SemaphoreType.DMA,
            pltpu.SemaphoreType.DMA,
            pltpu.SemaphoreType.DMA,
            pltpu.SemaphoreType.DMA,
            pltpu.SemaphoreType.DMA,
        ],
        compiler_params=pltpu.CompilerParams(use_tc_tiling_on_sc=False,
                                             needs_layout_passes=False),
    )
    def k(sidx_h, timef_h, pjob_h, pmach_h, pseq_h, vrow_h, out_h,
          idx0, idx1, tb0, tb1, bufj0, bufj1, bufm0, bufm1, bufs0, bufs1,
          ob0, ob1, vbuf, sem0, sem1, semo0, semo1, semi0, semi1, semt0, semt1):
        wid = lax.axis_index("s") * NC + lax.axis_index("c")
        pltpu.sync_copy(vrow_h, vbuf)
        vregs = [vbuf[pl.ds(r * 16, 16)] for r in range(D // 16)]
        idx = (idx0, idx1)
        tbuf = (tb0, tb1)
        bufj = (bufj0, bufj1)
        bufm = (bufm0, bufm1)
        bufs = (bufs0, bufs1)
        outb = (ob0, ob1)
        sems = (sem0, sem1)
        semo = (semo0, semo1)
        semi = (semi0, semi1)
        semt = (semt0, semt1)
        grp0 = wid * GPW

        def prefetch_idx(s, g):
            pltpu.async_copy(sidx_h.at[:, pl.ds(grp0 + g * 2, 2), :],
                             idx[s], semi[s])

        def prefetch_t(s, g):
            pltpu.async_copy(timef_h.at[pl.ds(grp0 + g * 2, 2)],
                             tbuf[s], semt[s])

        def fire(s, g):
            pltpu.make_async_copy(sidx_h.at[:, pl.ds(grp0 + g * 2, 2), :],
                                  idx[s], semi[s]).wait()
            for j in range(2):
                dst = pl.ds(j * 128, 128)
                pltpu.async_copy(pjob_h.at[idx[s].at[0, j]],
                                 bufj[s].at[dst], sems[s])
                pltpu.async_copy(pmach_h.at[idx[s].at[1, j]],
                                 bufm[s].at[dst], sems[s])
                pltpu.async_copy(pseq_h.at[idx[s].at[2, j]],
                                 bufs[s].at[dst], sems[s])

        def drain(s):
            for j in range(2):
                dst = pl.ds(j * 128, 128)
                pltpu.make_async_copy(pjob_h.at[idx[s].at[0, j]],
                                      bufj[s].at[dst], sems[s]).wait()
                pltpu.make_async_copy(pmach_h.at[idx[s].at[1, j]],
                                      bufm[s].at[dst], sems[s]).wait()
                pltpu.make_async_copy(pseq_h.at[idx[s].at[2, j]],
                                      bufs[s].at[dst], sems[s]).wait()

        def combine_store(s, g):
            pltpu.make_async_copy(timef_h.at[pl.ds(grp0 + g * 2, 2)],
                                  tbuf[s], semt[s]).wait()

            @pl.when(g >= 2)
            def _():
                pltpu.make_async_copy(
                    outb[s], out_h.at[pl.ds(0, 256)], semo[s]).wait()

            @pl.loop(0, 16)
            def grp(gg):
                tw = tbuf[s][gg // 8, pl.ds((gg % 8) * 16, 16)]
                for t in range(16):
                    tok = gg * 16 + t
                    st = lax.gather(
                        tw, jnp.full((16, 1), t, jnp.int32),
                        lax.GatherDimensionNumbers(
                            offset_dims=(), collapsed_slice_dims=(0,),
                            start_index_map=(0,)),
                        slice_sizes=(1,),
                        mode=lax.GatherScatterMode.PROMISE_IN_BOUNDS)
                    for q in range(2):
                        xj = bufj[s][tok, pl.ds(q * 32, 32)]
                        xm = bufm[s][tok, pl.ds(q * 32, 32)]
                        xs = bufs[s][tok, pl.ds(q * 32, 32)]
                        aj, bj = plsc.unpack(
                            xj, format=plsc.PackFormat.INTERLEAVED)
                        am, bm_ = plsc.unpack(
                            xm, format=plsc.PackFormat.INTERLEAVED)
                        as_, bs_ = plsc.unpack(
                            xs, format=plsc.PackFormat.INTERLEAVED)
                        outb[s][tok, pl.ds(q * 32, 16)] = (
                            aj + am + as_ + st * vregs[2 * q])
                        outb[s][tok, pl.ds(q * 32 + 16, 16)] = (
                            bj + bm_ + bs_ + st * vregs[2 * q + 1])

            pltpu.async_copy(
                outb[s], out_h.at[pl.ds(wid * TPW + g * 256, 256)], semo[s])

        prefetch_idx(0, 0)
        prefetch_t(0, 0)
        fire(0, 0)
        prefetch_idx(1, 1)
        prefetch_t(1, 1)

        @pl.loop(0, NCHUNK, step=2)
        def outer(g):
            drain(0)
            fire(1, g + 1)

            @pl.when(g + 2 < NCHUNK)
            def _():
                prefetch_idx(0, g + 2)
            combine_store(0, g)

            @pl.when(g + 2 < NCHUNK)
            def _():
                prefetch_t(0, g + 2)
            drain(1)

            @pl.when(g + 2 < NCHUNK)
            def _():
                fire(0, g + 2)

            @pl.when(g + 3 < NCHUNK)
            def _():
                prefetch_idx(1, g + 3)
            combine_store(1, g + 1)

            @pl.when(g + 3 < NCHUNK)
            def _():
                prefetch_t(1, g + 3)

        pltpu.make_async_copy(ob0, out_h.at[pl.ds(0, 256)], semo0).wait()
        pltpu.make_async_copy(ob1, out_h.at[pl.ds(0, 256)], semo1).wait()

    return k(sidx, timef, pjob, pmach, pseq, vrow)


def kernel(job, machine, sequence, time, job_table, machine_table, seq_table,
           W_time, b_time, W_proj, b_proj):
    perm = np.empty(D, np.int32)
    for q_ in range(2):
        for i_ in range(16):
            perm[q_ * 32 + 2 * i_] = q_ * 32 + i_
            perm[q_ * 32 + 2 * i_ + 1] = q_ * 32 + 16 + i_
    W_perm = W_proj[:, perm]
    pjob = _project_job_table(job_table, W_perm)
    pmach, pseq, vrow = _project_small_tables(
        machine_table, seq_table, W_perm, W_proj, W_time, b_time,
        b_proj[perm])
    sidx = jnp.stack([
        job.reshape(N).astype(jnp.int32),
        machine.reshape(N).astype(jnp.int32),
        sequence.reshape(N).astype(jnp.int32),
    ]).reshape(3, N // 128, 128)
    timef = time.reshape(N // 128, 128).astype(jnp.float32)
    out = _sc_gather_combine(sidx, timef, pjob, pmach, pseq, vrow.reshape(D))
    return out.reshape(B, L, D)
